# baseline (speedup 1.0000x reference)
import functools

import jax
import jax.numpy as jnp
from jax import lax
from jax.experimental import pallas as pl
from jax.experimental.pallas import tpu as pltpu
from jax.experimental.pallas import tpu_sc as plsc

B, K, D = 64, 512, 256
BK = B * K


def _degrees(adj):
    s = ((adj[:, 0:128] + adj[:, 128:256]) + adj[:, 256:384]) + adj[:, 384:512]
    acc = s[:, 0:8]
    for t in range(1, 16):
        acc = acc + s[:, 8 * t:8 * t + 8]
    b4 = acc[:, 0:4] + acc[:, 4:8]
    b2 = b4[:, 0:2] + b4[:, 2:4]
    return b2[:, 0:1] + b2[:, 1:2]


def _order_tc_body(adj_ref, order_ref, rev_ref, gidx_ref):
    b = pl.program_id(0)
    adj = adj_ref[0]
    deg_col = _degrees(adj)
    ii = lax.broadcasted_iota(jnp.int32, (K, K), 0)
    jj = lax.broadcasted_iota(jnp.int32, (K, K), 1)
    eye = ii == jj
    deg_row = jnp.sum(jnp.where(eye, deg_col, 0.0), axis=0, keepdims=True)
    h = (deg_col > deg_row) | ((deg_col == deg_row) & (ii < jj))
    rank_row = jnp.sum(h.astype(jnp.int32), axis=0, keepdims=True)
    rev_ref[0] = rank_row
    rank_col = jnp.sum(jnp.where(eye, rank_row, 0), axis=1, keepdims=True)
    order_row = jnp.sum(jnp.where(rank_col == jj, ii, 0), axis=0,
                        keepdims=True)
    order_ref[0] = order_row
    gidx_ref[0] = order_row + b * K


_order_tc = pl.pallas_call(
    _order_tc_body,
    grid=(B,),
    in_specs=[pl.BlockSpec((1, K, K), lambda b: (b, 0, 0))],
    out_specs=[
        pl.BlockSpec((1, 1, K), lambda b: (b, 0, 0)),
        pl.BlockSpec((1, 1, K), lambda b: (b, 0, 0)),
        pl.BlockSpec((1, 1, K), lambda b: (b, 0, 0)),
    ],
    out_shape=[
        jax.ShapeDtypeStruct((B, 1, K), jnp.int32),
        jax.ShapeDtypeStruct((B, 1, K), jnp.int32),
        jax.ShapeDtypeStruct((B, 1, K), jnp.int32),
    ],
)


_NC, _NS = 2, 16
_NW = _NC * _NS
_ROWS_PER_W = BK // _NW
_CHUNK = 128
_NCHUNK = _ROWS_PER_W // _CHUNK

@functools.cache
def _gather_sc():
    mesh = plsc.VectorSubcoreMesh(core_axis_name="c", subcore_axis_name="s")

    @functools.partial(
        pl.kernel,
        out_type=jax.ShapeDtypeStruct((BK, D), jnp.float32),
        mesh=mesh,
        scratch_types=[
            pltpu.VMEM((_CHUNK,), jnp.int32),
            pltpu.VMEM((_CHUNK, D), jnp.float32),
            pltpu.SemaphoreType.DMA,
        ],
    )
    def gather(table_hbm, idx_hbm, out_hbm, idx_v, rows_v, sem):
        wid = lax.axis_index("s") * _NC + lax.axis_index("c")
        base = wid * _ROWS_PER_W
        for c in range(_NCHUNK):
            off = base + c * _CHUNK
            pltpu.sync_copy(idx_hbm.at[pl.ds(off, _CHUNK)], idx_v)
            pltpu.async_copy(table_hbm.at[idx_v], rows_v, sem).wait()
            pltpu.sync_copy(rows_v, out_hbm.at[pl.ds(off, _CHUNK)])

    return gather


def kernel(slots, adj):
    order3, rev3, gidx3 = _order_tc(adj)
    order = order3.reshape(B, K)
    reverse_order = rev3.reshape(B, K)
    gidx = gidx3.reshape(BK)
    ordered = _gather_sc()(slots.reshape(BK, D), gidx)
    return ordered.reshape(B, K, D), order, reverse_order


# speedup vs baseline: 1.5965x; 1.5965x over previous
import functools

import jax
import jax.numpy as jnp
from jax import lax
from jax.experimental import pallas as pl
from jax.experimental.pallas import tpu as pltpu
from jax.experimental.pallas import tpu_sc as plsc

B, K, D = 64, 512, 256
BK = B * K


def _degrees(adj):
    s = ((adj[:, 0:128] + adj[:, 128:256]) + adj[:, 256:384]) + adj[:, 384:512]
    t = s
    for k in range(1, 16):
        t = t + pltpu.roll(s, 128 - 8 * k, 1)
    h4 = t + pltpu.roll(t, 124, 1)
    h2 = h4 + pltpu.roll(h4, 126, 1)
    h1 = h2 + pltpu.roll(h2, 127, 1)
    return h1[:, 0:1]


def _order_tc_body(adj_ref, order_ref, rev_ref, gidx_ref):
    b = pl.program_id(0)
    adj = adj_ref[0]
    deg_col = _degrees(adj)
    ii = lax.broadcasted_iota(jnp.int32, (K, K), 0)
    jj = lax.broadcasted_iota(jnp.int32, (K, K), 1)
    eye = ii == jj
    deg_row = jnp.sum(jnp.where(eye, deg_col, 0.0), axis=0, keepdims=True)
    h = (deg_col > deg_row) | ((deg_col == deg_row) & (ii < jj))
    rank_row = jnp.sum(h.astype(jnp.int32), axis=0, keepdims=True)
    rev_ref[0] = rank_row
    rank_col = jnp.sum(jnp.where(eye, rank_row, 0), axis=1, keepdims=True)
    order_row = jnp.sum(jnp.where(rank_col == jj, ii, 0), axis=0,
                        keepdims=True)
    order_ref[0] = order_row
    gidx_ref[0] = order_row + b * K


_order_tc = pl.pallas_call(
    _order_tc_body,
    grid=(B,),
    in_specs=[pl.BlockSpec((1, K, K), lambda b: (b, 0, 0))],
    out_specs=[
        pl.BlockSpec((1, 1, K), lambda b: (b, 0, 0)),
        pl.BlockSpec((1, 1, K), lambda b: (b, 0, 0)),
        pl.BlockSpec((1, 1, K), lambda b: (b, 0, 0)),
    ],
    out_shape=[
        jax.ShapeDtypeStruct((B, 1, K), jnp.int32),
        jax.ShapeDtypeStruct((B, 1, K), jnp.int32),
        jax.ShapeDtypeStruct((B, 1, K), jnp.int32),
    ],
)


_NC, _NS = 2, 16
_NW = _NC * _NS
_ROWS_PER_W = BK // _NW
_CHUNK = 128
_NCHUNK = _ROWS_PER_W // _CHUNK


@functools.cache
def _gather_sc():
    mesh = plsc.VectorSubcoreMesh(core_axis_name="c", subcore_axis_name="s")

    @functools.partial(
        pl.kernel,
        out_type=jax.ShapeDtypeStruct((BK, D), jnp.float32),
        mesh=mesh,
        scratch_types=[
            pltpu.VMEM((_NCHUNK, _CHUNK), jnp.int32),
            pltpu.VMEM((_CHUNK, D), jnp.float32),
            pltpu.VMEM((_CHUNK, D), jnp.float32),
            pltpu.SemaphoreType.DMA,
            pltpu.SemaphoreType.DMA,
            pltpu.SemaphoreType.DMA,
            pltpu.SemaphoreType.DMA,
        ],
    )
    def gather(table_hbm, idx_hbm, out_hbm, idx_v, buf0, buf1,
               sg0, sg1, sw0, sw1):
        wid = lax.axis_index("s") * _NC + lax.axis_index("c")
        base = wid * _ROWS_PER_W
        pltpu.sync_copy(idx_hbm.at[wid], idx_v)
        bufs = (buf0, buf1)
        gsems = (sg0, sg1)
        wsems = (sw0, sw1)
        ghs = [None] * _NCHUNK
        whs = [None] * _NCHUNK
        for c in range(_NCHUNK):
            if c >= 2:
                whs[c - 2].wait()
            ghs[c] = pltpu.async_copy(
                table_hbm.at[idx_v.at[c]], bufs[c % 2], gsems[c % 2])
            if c >= 1:
                ghs[c - 1].wait()
                whs[c - 1] = pltpu.async_copy(
                    bufs[(c - 1) % 2],
                    out_hbm.at[pl.ds(base + (c - 1) * _CHUNK, _CHUNK)],
                    wsems[(c - 1) % 2])
        ghs[_NCHUNK - 1].wait()
        whs[_NCHUNK - 1] = pltpu.async_copy(
            bufs[(_NCHUNK - 1) % 2],
            out_hbm.at[pl.ds(base + (_NCHUNK - 1) * _CHUNK, _CHUNK)],
            wsems[(_NCHUNK - 1) % 2])
        whs[_NCHUNK - 2].wait()
        whs[_NCHUNK - 1].wait()

    return gather


def kernel(slots, adj):
    order3, rev3, gidx3 = _order_tc(adj)
    order = order3.reshape(B, K)
    reverse_order = rev3.reshape(B, K)
    gidx = gidx3.reshape(_NW, _NCHUNK, _CHUNK)
    ordered = _gather_sc()(slots.reshape(BK, D), gidx)
    return ordered.reshape(B, K, D), order, reverse_order


# speedup vs baseline: 1.5969x; 1.0003x over previous
import functools

import jax
import jax.numpy as jnp
from jax import lax
from jax.experimental import pallas as pl
from jax.experimental.pallas import tpu as pltpu
from jax.experimental.pallas import tpu_sc as plsc

B, K, D = 64, 512, 256
BK = B * K


def _degrees_row(adj):
    s = ((adj[:, 0:128] + adj[:, 128:256]) + adj[:, 256:384]) + adj[:, 384:512]
    st = jnp.swapaxes(s, 0, 1)
    acc = st[0:8, :]
    for t in range(1, 16):
        acc = acc + st[8 * t:8 * t + 8, :]
    h4 = acc[0:4, :] + acc[4:8, :]
    h2 = h4[0:2, :] + h4[2:4, :]
    return h2[0:1, :] + h2[1:2, :]


def _order_tc_body(adj_ref, order_ref, rev_ref, gidx_ref):
    b = pl.program_id(0)
    adj = adj_ref[0]
    deg_row = _degrees_row(adj)
    ii = lax.broadcasted_iota(jnp.int32, (K, K), 0)
    jj = lax.broadcasted_iota(jnp.int32, (K, K), 1)
    eye = ii == jj
    deg_col = jnp.sum(jnp.where(eye, deg_row, 0.0), axis=1, keepdims=True)
    h = (deg_col > deg_row) | ((deg_col == deg_row) & (ii < jj))
    rank_row = jnp.sum(h.astype(jnp.int32), axis=0, keepdims=True)
    rev_ref[0] = rank_row
    rank_col = jnp.sum(jnp.where(eye, rank_row, 0), axis=1, keepdims=True)
    order_row = jnp.sum(jnp.where(rank_col == jj, ii, 0), axis=0,
                        keepdims=True)
    order_ref[0] = order_row
    gidx_ref[0] = order_row + b * K


_order_tc = pl.pallas_call(
    _order_tc_body,
    grid=(B,),
    in_specs=[pl.BlockSpec((1, K, K), lambda b: (b, 0, 0))],
    out_specs=[
        pl.BlockSpec((1, 1, K), lambda b: (b, 0, 0)),
        pl.BlockSpec((1, 1, K), lambda b: (b, 0, 0)),
        pl.BlockSpec((1, 1, K), lambda b: (b, 0, 0)),
    ],
    out_shape=[
        jax.ShapeDtypeStruct((B, 1, K), jnp.int32),
        jax.ShapeDtypeStruct((B, 1, K), jnp.int32),
        jax.ShapeDtypeStruct((B, 1, K), jnp.int32),
    ],
)


_NC, _NS = 2, 16
_NW = _NC * _NS
_ROWS_PER_W = BK // _NW
_CHUNK = 128
_NCHUNK = _ROWS_PER_W // _CHUNK


@functools.cache
def _gather_sc():
    mesh = plsc.VectorSubcoreMesh(core_axis_name="c", subcore_axis_name="s")

    @functools.partial(
        pl.kernel,
        out_type=jax.ShapeDtypeStruct((BK, D), jnp.float32),
        mesh=mesh,
        scratch_types=[
            pltpu.VMEM((_NCHUNK, _CHUNK), jnp.int32),
            pltpu.VMEM((_CHUNK, D), jnp.float32),
            pltpu.VMEM((_CHUNK, D), jnp.float32),
            pltpu.SemaphoreType.DMA,
            pltpu.SemaphoreType.DMA,
            pltpu.SemaphoreType.DMA,
            pltpu.SemaphoreType.DMA,
        ],
    )
    def gather(table_hbm, idx_hbm, out_hbm, idx_v, buf0, buf1,
               sg0, sg1, sw0, sw1):
        wid = lax.axis_index("s") * _NC + lax.axis_index("c")
        base = wid * _ROWS_PER_W
        pltpu.sync_copy(idx_hbm.at[wid], idx_v)
        bufs = (buf0, buf1)
        gsems = (sg0, sg1)
        wsems = (sw0, sw1)
        ghs = [None] * _NCHUNK
        whs = [None] * _NCHUNK
        for c in range(_NCHUNK):
            if c >= 2:
                whs[c - 2].wait()
            ghs[c] = pltpu.async_copy(
                table_hbm.at[idx_v.at[c]], bufs[c % 2], gsems[c % 2])
            if c >= 1:
                ghs[c - 1].wait()
                whs[c - 1] = pltpu.async_copy(
                    bufs[(c - 1) % 2],
                    out_hbm.at[pl.ds(base + (c - 1) * _CHUNK, _CHUNK)],
                    wsems[(c - 1) % 2])
        ghs[_NCHUNK - 1].wait()
        whs[_NCHUNK - 1] = pltpu.async_copy(
            bufs[(_NCHUNK - 1) % 2],
            out_hbm.at[pl.ds(base + (_NCHUNK - 1) * _CHUNK, _CHUNK)],
            wsems[(_NCHUNK - 1) % 2])
        whs[_NCHUNK - 2].wait()
        whs[_NCHUNK - 1].wait()

    return gather


def kernel(slots, adj):
    order3, rev3, gidx3 = _order_tc(adj)
    order = order3.reshape(B, K)
    reverse_order = rev3.reshape(B, K)
    gidx = gidx3.reshape(_NW, _NCHUNK, _CHUNK)
    ordered = _gather_sc()(slots.reshape(BK, D), gidx)
    return ordered.reshape(B, K, D), order, reverse_order
